# Initial kernel scaffold; baseline (speedup 1.0000x reference)
import jax
import jax.numpy as jnp
from jax.experimental import pallas as pl


def kernel(x, gate_w, w1, w2, w3):
    raise NotImplementedError("write your pallas kernel here")


# speedup vs baseline: 1.1623x; 1.1623x over previous
import functools

import jax
import jax.numpy as jnp
from jax.experimental import pallas as pl
from jax.experimental.pallas import tpu as pltpu

D_MODEL = 1024
D_FF = 2816
N_EXP = 8
TOPK = 2
EPAD = 128


def _routing_body(x_ref, gw_ref, cmb_ref):
    x = x_ref[...]
    logits = jax.lax.dot_general(
        x, gw_ref[...], (((1,), (1,)), ((), ())),
        preferred_element_type=jnp.float32,
    )
    T = logits.shape[0]
    lane = jax.lax.broadcasted_iota(jnp.int32, (T, EPAD), 1)
    neg = jnp.float32(-1e30)
    logits = jnp.where(lane < N_EXP, logits, neg)
    m = jnp.max(logits, axis=1, keepdims=True)
    ex = jnp.where(lane < N_EXP, jnp.exp(logits - m), 0.0)
    probs = ex / jnp.sum(ex, axis=1, keepdims=True)
    p0 = jnp.max(probs, axis=1, keepdims=True)
    i0 = jnp.min(jnp.where(probs == p0, lane, EPAD), axis=1, keepdims=True)
    probs2 = jnp.where(lane == i0, -1.0, probs)
    p1 = jnp.max(probs2, axis=1, keepdims=True)
    i1 = jnp.min(jnp.where(probs2 == p1, lane, EPAD), axis=1, keepdims=True)
    denom = p0 + p1
    cmb = jnp.where(lane == i0, p0 / denom, 0.0) + jnp.where(
        lane == i1, p1 / denom, 0.0
    )
    cmb_ref[...] = cmb


def _routing(x2d, gate_w):
    T = x2d.shape[0]
    gwp = jnp.zeros((EPAD, D_MODEL), jnp.float32).at[:N_EXP].set(gate_w)
    return pl.pallas_call(
        _routing_body,
        out_shape=jax.ShapeDtypeStruct((T, EPAD), jnp.float32),
    )(x2d, gwp)


def _dense_body(x_ref, w1_ref, w3_ref, w2_ref, c_ref, out_ref):
    e = pl.program_id(0)
    f = pl.program_id(1)
    x = x_ref[...]
    g = jax.lax.dot_general(
        x, w1_ref[0], (((1,), (1,)), ((), ())),
        preferred_element_type=jnp.float32,
    )
    v = jax.lax.dot_general(
        x, w3_ref[0], (((1,), (1,)), ((), ())),
        preferred_element_type=jnp.float32,
    )
    h = (g * jax.lax.logistic(g) * v).astype(jnp.bfloat16)
    oc = jax.lax.dot_general(
        h, w2_ref[0], (((1,), (1,)), ((), ())),
        preferred_element_type=jnp.float32,
    )
    contrib = oc * c_ref[0]

    @pl.when(jnp.logical_and(e == 0, f == 0))
    def _():
        out_ref[...] = contrib

    @pl.when(jnp.logical_or(e != 0, f != 0))
    def _():
        out_ref[...] += contrib


def _dense_moe(x2d, cmb, w1, w2, w3, ffc=1408):
    T = x2d.shape[0]
    nf = D_FF // ffc
    xb = x2d.astype(jnp.bfloat16)
    w1b = w1.astype(jnp.bfloat16)
    w2b = w2.astype(jnp.bfloat16)
    w3b = w3.astype(jnp.bfloat16)
    cmb8 = jnp.transpose(cmb[:, :N_EXP])[:, :, None]
    return pl.pallas_call(
        _dense_body,
        grid=(N_EXP, nf),
        in_specs=[
            pl.BlockSpec((T, D_MODEL), lambda e, f: (0, 0)),
            pl.BlockSpec((1, ffc, D_MODEL), lambda e, f: (e, f, 0)),
            pl.BlockSpec((1, ffc, D_MODEL), lambda e, f: (e, f, 0)),
            pl.BlockSpec((1, D_MODEL, ffc), lambda e, f: (e, 0, f)),
            pl.BlockSpec((1, T, 1), lambda e, f: (e, 0, 0)),
        ],
        out_specs=pl.BlockSpec((T, D_MODEL), lambda e, f: (0, 0)),
        out_shape=jax.ShapeDtypeStruct((T, D_MODEL), jnp.float32),
        compiler_params=pltpu.CompilerParams(
            dimension_semantics=("arbitrary", "arbitrary"),
        ),
    )(xb, w1b, w3b, w2b, cmb8)


@jax.jit
def kernel(x, gate_w, w1, w2, w3):
    B, S, D = x.shape
    x2d = x.reshape(-1, D)
    cmb = _routing(x2d, gate_w)
    out = _dense_moe(x2d, cmb, w1, w2, w3)
    return out.reshape(B, S, D)
